# baseline (device time: 24398 ns/iter reference)
import jax
import jax.numpy as jnp
from jax import lax
from jax.experimental import pallas as pl
from jax.experimental.pallas import tpu as pltpu

N_DEV = 32
EPS = 1e-5
NH = 2
NCH = 4


def kernel(x, t_emb, W_scale, W_shift):
    b, s, c_per = x.shape
    c_global = c_per * N_DEV
    nstat = 2 * b
    sh = s // NH
    sc = s // NCH

    def body(
        x_ref, t_ref, ws_ref, wsh_ref, out_ref,
        xv_ref, loc_ref, red1_ref, red2_ref, bufx_ref, bufy_ref, bufz_ref,
        cp_sems, sx_sems, rx_sems, sy_sems, ry_sems, sz_sems, rz_sems,
    ):
        my = lax.axis_index("i")
        my_z = my // 8
        q = my - 8 * my_z
        my_y = q // 2
        xbit = lax.rem(q, 2)
        my_x = jnp.where(lax.rem(my_y, 2) == 1, 1 - xbit, xbit)

        def dev_of(y, z):
            qq = 2 * y + jnp.where(lax.rem(y, 2) == 1, 1 - my_x, my_x)
            return 8 * z + qq

        x_partner = 8 * my_z + (q + 1 - 2 * xbit)
        y_peers = [dev_of(lax.rem(my_y + k, 4), my_z) for k in range(1, 4)]
        z_peers = [8 * lax.rem(my_z + k, 4) + q for k in range(1, 4)]

        barrier_sem = pltpu.get_barrier_semaphore()
        for peer in [x_partner] + y_peers + z_peers:
            pl.semaphore_signal(
                barrier_sem, inc=1,
                device_id=(peer,), device_id_type=pl.DeviceIdType.MESH,
            )

        rows = [slice(i * sc, (i + 1) * sc) for i in range(NCH)]
        copies = []
        for i in range(NCH):
            cp = pltpu.make_async_copy(
                x_ref.at[:, rows[i], :], xv_ref.at[:, rows[i], :],
                cp_sems.at[i],
            )
            cp.start()
            copies.append(cp)

        t = t_ref[...]
        scale = jnp.dot(t, ws_ref[...], preferred_element_type=jnp.float32)
        shift = jnp.dot(t, wsh_ref[...], preferred_element_type=jnp.float32)
        one_scale = 1.0 + scale

        cols = [slice(h * sh, (h + 1) * sh) for h in range(NH)]
        cpr = NCH // NH
        waits = []
        dxs = []

        for h in range(NH):
            for i in range(h * cpr, (h + 1) * cpr):
                copies[i].wait()
                xi = xv_ref[:, rows[i], :]
                ssum = jnp.sum(xi, axis=-1)
                ssq = jnp.sum(xi * xi, axis=-1)
                loc_ref[:, rows[i]] = jnp.concatenate([ssum, ssq], axis=0)
            if h == 0:
                pl.semaphore_wait(barrier_sem, 7)
            dx = pltpu.make_async_remote_copy(
                src_ref=loc_ref.at[:, cols[h]],
                dst_ref=bufx_ref.at[:, cols[h]],
                send_sem=sx_sems.at[h],
                recv_sem=rx_sems.at[h],
                device_id=(x_partner,),
                device_id_type=pl.DeviceIdType.MESH,
            )
            dx.start()
            dxs.append(dx)
            waits.append(dx)

        for h in range(NH):
            dxs[h].wait_recv()
            red1_ref[:, cols[h]] = (
                loc_ref[:, cols[h]] + bufx_ref[:, cols[h]]
            )
            for k in range(1, 4):
                d = pltpu.make_async_remote_copy(
                    src_ref=red1_ref.at[:, cols[h]],
                    dst_ref=bufy_ref.at[my_y, :, cols[h]],
                    send_sem=sy_sems.at[h, k],
                    recv_sem=ry_sems.at[h, my_y],
                    device_id=(y_peers[k - 1],),
                    device_id_type=pl.DeviceIdType.MESH,
                )
                d.start()
                waits.append(d)
            bufy_ref[pl.ds(my_y, 1), :, cols[h]] = red1_ref[:, cols[h]][None]

        for h in range(NH):
            for k in range(1, 4):
                src_y = lax.rem(my_y + k, 4)
                recv = pltpu.make_async_remote_copy(
                    src_ref=red1_ref.at[:, cols[h]],
                    dst_ref=bufy_ref.at[src_y, :, cols[h]],
                    send_sem=sy_sems.at[h, 0],
                    recv_sem=ry_sems.at[h, src_y],
                    device_id=(x_partner,),
                    device_id_type=pl.DeviceIdType.MESH,
                )
                recv.wait_recv()
            red2_ref[:, cols[h]] = jnp.sum(bufy_ref[:, :, cols[h]], axis=0)
            for k in range(1, 4):
                d = pltpu.make_async_remote_copy(
                    src_ref=red2_ref.at[:, cols[h]],
                    dst_ref=bufz_ref.at[my_z, :, cols[h]],
                    send_sem=sz_sems.at[h, k],
                    recv_sem=rz_sems.at[h, my_z],
                    device_id=(z_peers[k - 1],),
                    device_id_type=pl.DeviceIdType.MESH,
                )
                d.start()
                waits.append(d)
            bufz_ref[pl.ds(my_z, 1), :, cols[h]] = red2_ref[:, cols[h]][None]

        for h in range(NH):
            for k in range(1, 4):
                src_z = lax.rem(my_z + k, 4)
                recv = pltpu.make_async_remote_copy(
                    src_ref=red2_ref.at[:, cols[h]],
                    dst_ref=bufz_ref.at[src_z, :, cols[h]],
                    send_sem=sz_sems.at[h, 0],
                    recv_sem=rz_sems.at[h, src_z],
                    device_id=(x_partner,),
                    device_id_type=pl.DeviceIdType.MESH,
                )
                recv.wait_recv()
            tot = jnp.sum(bufz_ref[:, :, cols[h]], axis=0)
            mean = tot[:b] / c_global
            ex2 = tot[b:] / c_global
            var = ex2 - mean * mean
            inv = lax.rsqrt(var + EPS)

            xh = xv_ref[:, cols[h], :]
            hnorm = (xh - mean[..., None]) * inv[..., None]
            out = hnorm * one_scale[:, None, :] + shift[:, None, :]
            out_ref[:, cols[h], :] = out.astype(jnp.bfloat16)

        for d in waits:
            d.wait_send()

    return pl.pallas_call(
        body,
        out_shape=jax.ShapeDtypeStruct((b, s, c_per), jnp.bfloat16),
        in_specs=[
            pl.BlockSpec(memory_space=pl.ANY),
            pl.BlockSpec(memory_space=pltpu.VMEM),
            pl.BlockSpec(memory_space=pltpu.VMEM),
            pl.BlockSpec(memory_space=pltpu.VMEM),
        ],
        out_specs=pl.BlockSpec(memory_space=pltpu.VMEM),
        scratch_shapes=[
            pltpu.VMEM((b, s, c_per), jnp.float32),
            pltpu.VMEM((nstat, s), jnp.float32),
            pltpu.VMEM((nstat, s), jnp.float32),
            pltpu.VMEM((nstat, s), jnp.float32),
            pltpu.VMEM((nstat, s), jnp.float32),
            pltpu.VMEM((4, nstat, s), jnp.float32),
            pltpu.VMEM((4, nstat, s), jnp.float32),
            pltpu.SemaphoreType.DMA((NCH,)),
            pltpu.SemaphoreType.DMA((NH,)),
            pltpu.SemaphoreType.DMA((NH,)),
            pltpu.SemaphoreType.DMA((NH, 4)),
            pltpu.SemaphoreType.DMA((NH, 4)),
            pltpu.SemaphoreType.DMA((NH, 4)),
            pltpu.SemaphoreType.DMA((NH, 4)),
        ],
        compiler_params=pltpu.CompilerParams(collective_id=0),
    )(x, t_emb, W_scale, W_shift)


# device time: 24052 ns/iter; 1.0144x vs baseline; 1.0144x over previous
import jax
import jax.numpy as jnp
from jax import lax
from jax.experimental import pallas as pl
from jax.experimental.pallas import tpu as pltpu

N_DEV = 32
EPS = 1e-5
NH = 2


def kernel(x, t_emb, W_scale, W_shift):
    b, s, c_per = x.shape
    c_global = c_per * N_DEV
    nstat = 2 * b
    sh = s // NH

    def body(
        x_ref, t_ref, ws_ref, wsh_ref, out_ref,
        loc_ref, red1_ref, red2_ref, bufx_ref, bufy_ref, bufz_ref,
        sx_sems, rx_sems, sy_sems, ry_sems, sz_sems, rz_sems,
    ):
        my = lax.axis_index("i")
        my_z = my // 8
        q = my - 8 * my_z
        my_y = q // 2
        xbit = lax.rem(q, 2)
        my_x = jnp.where(lax.rem(my_y, 2) == 1, 1 - xbit, xbit)

        def dev_of(y, z):
            qq = 2 * y + jnp.where(lax.rem(y, 2) == 1, 1 - my_x, my_x)
            return 8 * z + qq

        x_partner = 8 * my_z + (q + 1 - 2 * xbit)
        y_peers = [dev_of(lax.rem(my_y + k, 4), my_z) for k in range(1, 4)]
        z_peers = [8 * lax.rem(my_z + k, 4) + q for k in range(1, 4)]

        barrier_sem = pltpu.get_barrier_semaphore()
        for peer in [x_partner] + y_peers + z_peers:
            pl.semaphore_signal(
                barrier_sem, inc=1,
                device_id=(peer,), device_id_type=pl.DeviceIdType.MESH,
            )

        xf = x_ref[...].astype(jnp.float32)
        ssum = jnp.sum(xf, axis=-1)
        ssq = jnp.sum(xf * xf, axis=-1)
        loc = jnp.concatenate([ssum, ssq], axis=0)
        loc_ref[...] = loc

        pl.semaphore_wait(barrier_sem, 7)

        cols = [slice(h * sh, (h + 1) * sh) for h in range(NH)]
        waits = []

        dxs = []
        for h in range(NH):
            dx = pltpu.make_async_remote_copy(
                src_ref=loc_ref.at[:, cols[h]],
                dst_ref=bufx_ref.at[:, cols[h]],
                send_sem=sx_sems.at[h],
                recv_sem=rx_sems.at[h],
                device_id=(x_partner,),
                device_id_type=pl.DeviceIdType.MESH,
            )
            dx.start()
            dxs.append(dx)
            waits.append(dx)

        t = t_ref[...]
        scale = jnp.dot(t, ws_ref[...], preferred_element_type=jnp.float32)
        shift = jnp.dot(t, wsh_ref[...], preferred_element_type=jnp.float32)

        for h in range(NH):
            dxs[h].wait_recv()
            red1_ref[:, cols[h]] = (
                loc_ref[:, cols[h]] + bufx_ref[:, cols[h]]
            )
            for k in range(1, 4):
                d = pltpu.make_async_remote_copy(
                    src_ref=red1_ref.at[:, cols[h]],
                    dst_ref=bufy_ref.at[my_y, :, cols[h]],
                    send_sem=sy_sems.at[h, k],
                    recv_sem=ry_sems.at[h, my_y],
                    device_id=(y_peers[k - 1],),
                    device_id_type=pl.DeviceIdType.MESH,
                )
                d.start()
                waits.append(d)
            bufy_ref[pl.ds(my_y, 1), :, cols[h]] = red1_ref[:, cols[h]][None]

        for h in range(NH):
            for k in range(1, 4):
                src_y = lax.rem(my_y + k, 4)
                recv = pltpu.make_async_remote_copy(
                    src_ref=red1_ref.at[:, cols[h]],
                    dst_ref=bufy_ref.at[src_y, :, cols[h]],
                    send_sem=sy_sems.at[h, 0],
                    recv_sem=ry_sems.at[h, src_y],
                    device_id=(x_partner,),
                    device_id_type=pl.DeviceIdType.MESH,
                )
                recv.wait_recv()
            red2_ref[:, cols[h]] = jnp.sum(bufy_ref[:, :, cols[h]], axis=0)
            for k in range(1, 4):
                d = pltpu.make_async_remote_copy(
                    src_ref=red2_ref.at[:, cols[h]],
                    dst_ref=bufz_ref.at[my_z, :, cols[h]],
                    send_sem=sz_sems.at[h, k],
                    recv_sem=rz_sems.at[h, my_z],
                    device_id=(z_peers[k - 1],),
                    device_id_type=pl.DeviceIdType.MESH,
                )
                d.start()
                waits.append(d)
            bufz_ref[pl.ds(my_z, 1), :, cols[h]] = red2_ref[:, cols[h]][None]

        one_scale = 1.0 + scale
        for h in range(NH):
            for k in range(1, 4):
                src_z = lax.rem(my_z + k, 4)
                recv = pltpu.make_async_remote_copy(
                    src_ref=red2_ref.at[:, cols[h]],
                    dst_ref=bufz_ref.at[src_z, :, cols[h]],
                    send_sem=sz_sems.at[h, 0],
                    recv_sem=rz_sems.at[h, src_z],
                    device_id=(x_partner,),
                    device_id_type=pl.DeviceIdType.MESH,
                )
                recv.wait_recv()
            tot = jnp.sum(bufz_ref[:, :, cols[h]], axis=0)
            mean = tot[:b] / c_global
            ex2 = tot[b:] / c_global
            var = ex2 - mean * mean
            inv = lax.rsqrt(var + EPS)

            xh = xf[:, cols[h], :]
            hnorm = (xh - mean[..., None]) * inv[..., None]
            out = hnorm * one_scale[:, None, :] + shift[:, None, :]
            out_ref[:, cols[h], :] = out.astype(jnp.bfloat16)

        for d in waits:
            d.wait_send()

    return pl.pallas_call(
        body,
        out_shape=jax.ShapeDtypeStruct((b, s, c_per), jnp.bfloat16),
        in_specs=[pl.BlockSpec(memory_space=pltpu.VMEM)] * 4,
        out_specs=pl.BlockSpec(memory_space=pltpu.VMEM),
        scratch_shapes=[
            pltpu.VMEM((nstat, s), jnp.float32),
            pltpu.VMEM((nstat, s), jnp.float32),
            pltpu.VMEM((nstat, s), jnp.float32),
            pltpu.VMEM((nstat, s), jnp.float32),
            pltpu.VMEM((4, nstat, s), jnp.float32),
            pltpu.VMEM((4, nstat, s), jnp.float32),
            pltpu.SemaphoreType.DMA((NH,)),
            pltpu.SemaphoreType.DMA((NH,)),
            pltpu.SemaphoreType.DMA((NH, 4)),
            pltpu.SemaphoreType.DMA((NH, 4)),
            pltpu.SemaphoreType.DMA((NH, 4)),
            pltpu.SemaphoreType.DMA((NH, 4)),
        ],
        compiler_params=pltpu.CompilerParams(collective_id=0),
    )(x, t_emb, W_scale, W_shift)
